# baseline (speedup 1.0000x reference)
import functools

import jax
import jax.numpy as jnp
from jax import lax
from jax.experimental import pallas as pl
from jax.experimental.pallas import tpu as pltpu
from jax.experimental.pallas import tpu_sc as plsc

NC = 2
NS = 16
L = 16
CHUNK = 128


@functools.lru_cache(maxsize=None)
def _build(B, D):
    assert D == L
    NW = NC * NS
    BPW = B // NW
    assert BPW % CHUNK == 0
    NCH = BPW // CHUNK

    mesh = plsc.VectorSubcoreMesh(
        core_axis_name="c", subcore_axis_name="s",
        num_cores=NC, num_subcores=NS)

    @functools.partial(
        pl.kernel,
        out_type=jax.ShapeDtypeStruct((B,), jnp.float32),
        mesh=mesh,
        compiler_params=pltpu.CompilerParams(
            needs_layout_passes=False, use_tc_tiling_on_sc=False),
        scratch_types=[
            pltpu.VMEM((BPW,), jnp.int32),
            pltpu.VMEM((BPW,), jnp.int32),
            pltpu.VMEM((D * BPW,), jnp.float32),
            pltpu.VMEM((D * BPW,), jnp.float32),
            pltpu.VMEM((D * BPW,), jnp.float32),
            pltpu.VMEM((D * BPW,), jnp.float32),
            pltpu.VMEM((BPW,), jnp.float32),
            pltpu.VMEM((BPW,), jnp.float32),
            pltpu.SemaphoreType.DMA,
            pltpu.SemaphoreType.DMA,
        ],
    )
    def hybrid_kernel(u_hbm, i_hbm, mod_u_hbm, mod_i_hbm, mem_u_hbm,
                      mem_i_hbm, alpha_hbm, out_hbm,
                      u_v, i_v, mu_v, mi_v, ku_v, ki_v, a_v, o_v, sem, sem_a):
        wid = lax.axis_index("s") * NC + lax.axis_index("c")
        base = wid * BPW

        pltpu.sync_copy(u_hbm.at[pl.ds(base, BPW)], u_v)
        pltpu.sync_copy(i_hbm.at[pl.ds(base, BPW)], i_v)

        a_copies = []
        for j in range(NCH):
            s = pl.ds(j * CHUNK, CHUNK)
            a_copies.append(
                pltpu.async_copy(alpha_hbm.at[u_v.at[s]], a_v.at[s], sem_a))

        copies = []
        for j in range(NCH):
            s = pl.ds(j * CHUNK, CHUNK)
            for d in range(D):
                t = pl.ds(d * BPW + j * CHUNK, CHUNK)
                copies.append(pltpu.async_copy(
                    mod_u_hbm.at[d].at[u_v.at[s]], mu_v.at[t], sem))
                copies.append(pltpu.async_copy(
                    mod_i_hbm.at[d].at[i_v.at[s]], mi_v.at[t], sem))
                copies.append(pltpu.async_copy(
                    mem_u_hbm.at[d].at[u_v.at[s]], ku_v.at[t], sem))
                copies.append(pltpu.async_copy(
                    mem_i_hbm.at[d].at[i_v.at[s]], ki_v.at[t], sem))
        for c in copies:
            c.wait()
        for c in a_copies:
            c.wait()

        def blk_body(b, carry):
            s = pl.ds(b * L, L)
            acc1 = jnp.zeros((L,), jnp.float32)
            acc2 = jnp.zeros((L,), jnp.float32)
            for d in range(D):
                t = pl.ds(d * BPW + b * L, L)
                acc1 = acc1 + mu_v[t] * mi_v[t]
                acc2 = acc2 + ku_v[t] * ki_v[t]
            a = a_v[s]
            o_v[s] = a * acc1 + (1.0 - a) * acc2
            return carry

        lax.fori_loop(0, BPW // L, blk_body, 0, unroll=False)
        pltpu.sync_copy(o_v, out_hbm.at[pl.ds(base, BPW)])

    return hybrid_kernel


def kernel(user_indices, item_indices, mod_user_emb, mod_item_emb,
           mem_user_emb, mem_item_emb, alpha_table):
    B = user_indices.shape[0]
    D = mod_user_emb.shape[1]
    return _build(B, D)(
        user_indices, item_indices,
        mod_user_emb.T, mod_item_emb.T, mem_user_emb.T, mem_item_emb.T,
        alpha_table.reshape(-1))


# speedup vs baseline: 3.4465x; 3.4465x over previous
import functools

import jax
import jax.numpy as jnp
from jax import lax
from jax.experimental import pallas as pl
from jax.experimental.pallas import tpu as pltpu
from jax.experimental.pallas import tpu_sc as plsc

NC = 2
NS = 16
L = 16
CHUNK = 128


@functools.lru_cache(maxsize=None)
def _build(B, D):
    assert D == L
    NW = NC * NS
    BPW = B // NW
    assert BPW % CHUNK == 0
    NCH = BPW // CHUNK

    mesh = plsc.VectorSubcoreMesh(
        core_axis_name="c", subcore_axis_name="s",
        num_cores=NC, num_subcores=NS)

    @functools.partial(
        pl.kernel,
        out_type=jax.ShapeDtypeStruct((B,), jnp.float32),
        mesh=mesh,
        compiler_params=pltpu.CompilerParams(
            needs_layout_passes=False, use_tc_tiling_on_sc=False),
        scratch_types=[
            pltpu.VMEM((BPW,), jnp.int32),
            pltpu.VMEM((BPW,), jnp.int32),
            pltpu.VMEM((BPW, L), jnp.float32),
            pltpu.VMEM((BPW, L), jnp.float32),
            pltpu.VMEM((BPW, L), jnp.float32),
            pltpu.VMEM((BPW, L), jnp.float32),
            pltpu.VMEM((BPW,), jnp.float32),
            pltpu.VMEM((BPW,), jnp.float32),
            pltpu.SemaphoreType.DMA,
        ],
    )
    def hybrid_kernel(u_hbm, i_hbm, mod_u_hbm, mod_i_hbm, mem_u_hbm,
                      mem_i_hbm, alpha_hbm, out_hbm,
                      u_v, i_v, mu_v, mi_v, ku_v, ki_v, a_v, o_v, sem):
        wid = lax.axis_index("s") * NC + lax.axis_index("c")
        base = wid * BPW

        pltpu.sync_copy(u_hbm.at[pl.ds(base, BPW)], u_v)
        pltpu.sync_copy(i_hbm.at[pl.ds(base, BPW)], i_v)

        copies = []
        for j in range(NCH):
            s = pl.ds(j * CHUNK, CHUNK)
            copies.append(pltpu.async_copy(mod_u_hbm.at[u_v.at[s]], mu_v.at[s], sem))
            copies.append(pltpu.async_copy(mod_i_hbm.at[i_v.at[s]], mi_v.at[s], sem))
            copies.append(pltpu.async_copy(mem_u_hbm.at[u_v.at[s]], ku_v.at[s], sem))
            copies.append(pltpu.async_copy(mem_i_hbm.at[i_v.at[s]], ki_v.at[s], sem))
            copies.append(pltpu.async_copy(alpha_hbm.at[u_v.at[s]], a_v.at[s], sem))
        for c in copies:
            c.wait()

        iota = lax.iota(jnp.int32, L)

        def blk_body(blk, carry):
            rows = blk * L + iota
            acc1 = jnp.zeros((L,), jnp.float32)
            acc2 = jnp.zeros((L,), jnp.float32)
            for d in range(D):
                dcol = jnp.full((L,), d, jnp.int32)
                acc1 = acc1 + (plsc.load_gather(mu_v, [rows, dcol])
                               * plsc.load_gather(mi_v, [rows, dcol]))
                acc2 = acc2 + (plsc.load_gather(ku_v, [rows, dcol])
                               * plsc.load_gather(ki_v, [rows, dcol]))
            a = a_v[pl.ds(blk * L, L)]
            o_v[pl.ds(blk * L, L)] = a * acc1 + (1.0 - a) * acc2
            return carry

        lax.fori_loop(0, BPW // L, blk_body, 0, unroll=False)
        pltpu.sync_copy(o_v, out_hbm.at[pl.ds(base, BPW)])

    return hybrid_kernel


def kernel(user_indices, item_indices, mod_user_emb, mod_item_emb,
           mem_user_emb, mem_item_emb, alpha_table):
    B = user_indices.shape[0]
    D = mod_user_emb.shape[1]
    return _build(B, D)(user_indices, item_indices, mod_user_emb,
                        mod_item_emb, mem_user_emb, mem_item_emb,
                        alpha_table.reshape(-1))
